# baseline (speedup 1.0000x reference)
import functools

import jax
import jax.numpy as jnp
from jax import lax
from jax.experimental import pallas as pl
from jax.experimental.pallas import tpu as pltpu
from jax.experimental.pallas import tpu_sc as plsc

VOCAB = 100000
EMBED = 64
HIDDEN = 512
B = 16
T = 20
SEQ = T + 1
SEQ_PAD = 24
NW = 32
PER_W = 16
IDX_PAD = NW * PER_W
V_BLK = 4096
N_VBLK = pl.cdiv(VOCAB, V_BLK)

_NT = (((1,), (1,)), ((), ()))


def _sc_gather(table, idx_pad):
    mesh = plsc.VectorSubcoreMesh(core_axis_name="c", subcore_axis_name="s")

    @functools.partial(
        pl.kernel,
        mesh=mesh,
        out_type=jax.ShapeDtypeStruct((IDX_PAD, EMBED), jnp.float32),
        scratch_types=[
            pltpu.VMEM((PER_W,), jnp.int32),
            pltpu.VMEM((PER_W, EMBED), jnp.float32),
            pltpu.SemaphoreType.DMA,
        ],
        compiler_params=pltpu.CompilerParams(use_tc_tiling_on_sc=False),
    )
    def k(table_hbm, idx_hbm, out_hbm, idx_v, rows_v, sem):
        wid = lax.axis_index("s") * 2 + lax.axis_index("c")
        base = wid * PER_W
        pltpu.sync_copy(idx_hbm.at[pl.ds(base, PER_W)], idx_v)
        pltpu.async_copy(table_hbm.at[idx_v], rows_v, sem).wait()
        pltpu.sync_copy(rows_v, out_hbm.at[pl.ds(base, PER_W)])

    return k(table, idx_pad)


def _fused_body(x_ref, wih_ref, whh_ref, bi_ref, bh_ref, wfc_ref, bfc_ref,
                o_ref, hs_scr):
    @pl.when(pl.program_id(0) == 0)
    def _lstm():
        bias = bi_ref[...] + bh_ref[...]

        def step(b, carry):
            h, c = carry
            x = x_ref[pl.ds(b * SEQ_PAD, SEQ_PAD), :]
            gates = (
                lax.dot_general(x, wih_ref[...], _NT,
                                preferred_element_type=jnp.float32)
                + lax.dot_general(h, whh_ref[...], _NT,
                                  preferred_element_type=jnp.float32)
                + bias
            )
            i = jax.nn.sigmoid(gates[:, :HIDDEN])
            f = jax.nn.sigmoid(gates[:, HIDDEN : 2 * HIDDEN])
            g = jnp.tanh(gates[:, 2 * HIDDEN : 3 * HIDDEN])
            o = jax.nn.sigmoid(gates[:, 3 * HIDDEN :])
            c = f * c + i * g
            h = o * jnp.tanh(c)
            hs_scr[pl.ds(b * SEQ_PAD, SEQ_PAD), :] = h
            return (h, c)

        init = (
            jnp.zeros((SEQ_PAD, HIDDEN), jnp.float32),
            jnp.zeros((SEQ_PAD, HIDDEN), jnp.float32),
        )
        lax.fori_loop(0, B, step, init)

    a = hs_scr[...].astype(jnp.bfloat16)
    w = wfc_ref[...].astype(jnp.bfloat16)
    acc = lax.dot_general(a, w, _NT, preferred_element_type=jnp.float32)
    acc = acc + bfc_ref[...]
    o_ref[...] = acc.reshape(B, SEQ_PAD, acc.shape[-1])[:, :SEQ, :]


def kernel(features, captions, emb_table, W_ih, W_hh, b_ih, b_hh, W_fc, b_fc):
    idx = captions.reshape(-1).astype(jnp.int32)
    idx_pad = jnp.pad(idx, (0, IDX_PAD - B * T))
    emb = _sc_gather(emb_table, idx_pad)[: B * T].reshape(B, T, EMBED)

    x = jnp.concatenate([features[:, None, :], emb], axis=1)
    x = jnp.pad(x, ((0, 0), (0, SEQ_PAD - SEQ), (0, 0)))
    x = x.reshape(B * SEQ_PAD, EMBED)

    c0 = lambda i: (0, 0)
    out = pl.pallas_call(
        _fused_body,
        grid=(N_VBLK,),
        in_specs=[
            pl.BlockSpec((B * SEQ_PAD, EMBED), c0),
            pl.BlockSpec((4 * HIDDEN, EMBED), c0),
            pl.BlockSpec((4 * HIDDEN, HIDDEN), c0),
            pl.BlockSpec((1, 4 * HIDDEN), c0),
            pl.BlockSpec((1, 4 * HIDDEN), c0),
            pl.BlockSpec((V_BLK, HIDDEN), lambda i: (i, 0)),
            pl.BlockSpec((1, V_BLK), lambda i: (0, i)),
        ],
        out_specs=pl.BlockSpec((B, SEQ, V_BLK), lambda i: (0, 0, i)),
        out_shape=jax.ShapeDtypeStruct((B, SEQ, VOCAB), jnp.float32),
        scratch_shapes=[pltpu.VMEM((B * SEQ_PAD, HIDDEN), jnp.float32)],
    )(x, W_ih, W_hh, b_ih.reshape(1, -1), b_hh.reshape(1, -1), W_fc,
      b_fc.reshape(1, VOCAB))
    return out


# speedup vs baseline: 1.0812x; 1.0812x over previous
import functools

import jax
import jax.numpy as jnp
from jax import lax
from jax.experimental import pallas as pl
from jax.experimental.pallas import tpu as pltpu
from jax.experimental.pallas import tpu_sc as plsc

VOCAB = 100000
EMBED = 64
HIDDEN = 512
B = 16
T = 20
SEQ = T + 1
SEQ_PAD = 24
NW = 32
PER_W = 16
IDX_PAD = NW * PER_W
V_BLK = 4096
N_VBLK = pl.cdiv(VOCAB, V_BLK)

_NT = (((1,), (1,)), ((), ()))


def _sc_gather(table, idx_pad):
    mesh = plsc.VectorSubcoreMesh(core_axis_name="c", subcore_axis_name="s")

    @functools.partial(
        pl.kernel,
        mesh=mesh,
        out_type=jax.ShapeDtypeStruct((IDX_PAD, EMBED), jnp.float32),
        scratch_types=[
            pltpu.VMEM((PER_W,), jnp.int32),
            pltpu.VMEM((PER_W, EMBED), jnp.float32),
            pltpu.SemaphoreType.DMA,
        ],
        compiler_params=pltpu.CompilerParams(use_tc_tiling_on_sc=True),
    )
    def k(table_hbm, idx_hbm, out_hbm, idx_v, rows_v, sem):
        wid = lax.axis_index("s") * 2 + lax.axis_index("c")
        base = wid * PER_W
        pltpu.sync_copy(idx_hbm.at[pl.ds(base, PER_W)], idx_v)
        iv = idx_v[...]
        copies = [
            pltpu.async_copy(table_hbm.at[iv[j]], rows_v.at[j], sem)
            for j in range(PER_W)
        ]
        for c in copies:
            c.wait()
        pltpu.sync_copy(rows_v, out_hbm.at[pl.ds(base, PER_W)])

    return k(table, idx_pad)


def _fused_body(x_ref, wih_ref, whh_ref, bi_ref, bh_ref, wfc_ref, bfc_ref,
                o_ref, hs_scr):
    @pl.when(pl.program_id(0) == 0)
    def _lstm():
        bias = bi_ref[...] + bh_ref[...]

        def step(b, carry):
            h, c = carry
            x = x_ref[pl.ds(b * SEQ_PAD, SEQ_PAD), :]
            gates = (
                lax.dot_general(x, wih_ref[...], _NT,
                                preferred_element_type=jnp.float32)
                + lax.dot_general(h, whh_ref[...], _NT,
                                  preferred_element_type=jnp.float32)
                + bias
            )
            i = jax.nn.sigmoid(gates[:, :HIDDEN])
            f = jax.nn.sigmoid(gates[:, HIDDEN : 2 * HIDDEN])
            g = jnp.tanh(gates[:, 2 * HIDDEN : 3 * HIDDEN])
            o = jax.nn.sigmoid(gates[:, 3 * HIDDEN :])
            c = f * c + i * g
            h = o * jnp.tanh(c)
            hs_scr[pl.ds(b * SEQ_PAD, SEQ_PAD), :] = h
            return (h, c)

        init = (
            jnp.zeros((SEQ_PAD, HIDDEN), jnp.float32),
            jnp.zeros((SEQ_PAD, HIDDEN), jnp.float32),
        )
        lax.fori_loop(0, B, step, init)

    a = hs_scr[...].astype(jnp.bfloat16)
    w = wfc_ref[...].astype(jnp.bfloat16)
    acc = lax.dot_general(a, w, _NT, preferred_element_type=jnp.float32)
    acc = acc + bfc_ref[...]
    o_ref[...] = acc.reshape(B, SEQ_PAD, acc.shape[-1])[:, :SEQ, :]


def kernel(features, captions, emb_table, W_ih, W_hh, b_ih, b_hh, W_fc, b_fc):
    idx = captions.reshape(-1).astype(jnp.int32)
    idx_pad = jnp.pad(idx, (0, IDX_PAD - B * T))
    emb = _sc_gather(emb_table, idx_pad)[: B * T].reshape(B, T, EMBED)

    x = jnp.concatenate([features[:, None, :], emb], axis=1)
    x = jnp.pad(x, ((0, 0), (0, SEQ_PAD - SEQ), (0, 0)))
    x = x.reshape(B * SEQ_PAD, EMBED)

    c0 = lambda i: (0, 0)
    out = pl.pallas_call(
        _fused_body,
        grid=(N_VBLK,),
        in_specs=[
            pl.BlockSpec((B * SEQ_PAD, EMBED), c0),
            pl.BlockSpec((4 * HIDDEN, EMBED), c0),
            pl.BlockSpec((4 * HIDDEN, HIDDEN), c0),
            pl.BlockSpec((1, 4 * HIDDEN), c0),
            pl.BlockSpec((1, 4 * HIDDEN), c0),
            pl.BlockSpec((V_BLK, HIDDEN), lambda i: (i, 0)),
            pl.BlockSpec((1, V_BLK), lambda i: (0, i)),
        ],
        out_specs=pl.BlockSpec((B, SEQ, V_BLK), lambda i: (0, 0, i)),
        out_shape=jax.ShapeDtypeStruct((B, SEQ, VOCAB), jnp.float32),
        scratch_shapes=[pltpu.VMEM((B * SEQ_PAD, HIDDEN), jnp.float32)],
    )(x, W_ih, W_hh, b_ih.reshape(1, -1), b_hh.reshape(1, -1), W_fc,
      b_fc.reshape(1, VOCAB))
    return out


# speedup vs baseline: 1.0814x; 1.0001x over previous
import functools

import jax
import jax.numpy as jnp
from jax import lax
from jax.experimental import pallas as pl
from jax.experimental.pallas import tpu as pltpu
from jax.experimental.pallas import tpu_sc as plsc

VOCAB = 100000
EMBED = 64
HIDDEN = 512
B = 16
T = 20
SEQ = T + 1
SEQ_PAD = 24
NW = 32
PER_W = 16
IDX_PAD = NW * PER_W
V_BLK = 6400
N_VBLK = pl.cdiv(VOCAB, V_BLK)

_NT = (((1,), (1,)), ((), ()))


def _sc_gather(table, idx_pad):
    mesh = plsc.VectorSubcoreMesh(core_axis_name="c", subcore_axis_name="s")

    @functools.partial(
        pl.kernel,
        mesh=mesh,
        out_type=jax.ShapeDtypeStruct((IDX_PAD, EMBED), jnp.float32),
        scratch_types=[
            pltpu.VMEM((PER_W,), jnp.int32),
            pltpu.VMEM((PER_W, EMBED), jnp.float32),
            pltpu.SemaphoreType.DMA,
        ],
        compiler_params=pltpu.CompilerParams(use_tc_tiling_on_sc=True),
    )
    def k(table_hbm, idx_hbm, out_hbm, idx_v, rows_v, sem):
        wid = lax.axis_index("s") * 2 + lax.axis_index("c")
        base = wid * PER_W
        pltpu.sync_copy(idx_hbm.at[pl.ds(base, PER_W)], idx_v)
        iv = idx_v[...]
        copies = [
            pltpu.async_copy(table_hbm.at[iv[j]], rows_v.at[j], sem)
            for j in range(PER_W)
        ]
        for c in copies:
            c.wait()
        pltpu.sync_copy(rows_v, out_hbm.at[pl.ds(base, PER_W)])

    return k(table, idx_pad)


def _fused_body(x_ref, wih_ref, whh_ref, bi_ref, bh_ref, wfc_ref, bfc_ref,
                o_ref, hs_scr):
    @pl.when(pl.program_id(0) == 0)
    def _lstm():
        bias = bi_ref[...] + bh_ref[...]

        def step(b, carry):
            h, c = carry
            x = x_ref[pl.ds(b * SEQ_PAD, SEQ_PAD), :]
            gates = (
                lax.dot_general(x, wih_ref[...], _NT,
                                preferred_element_type=jnp.float32)
                + lax.dot_general(h, whh_ref[...], _NT,
                                  preferred_element_type=jnp.float32)
                + bias
            )
            i = jax.nn.sigmoid(gates[:, :HIDDEN])
            f = jax.nn.sigmoid(gates[:, HIDDEN : 2 * HIDDEN])
            g = jnp.tanh(gates[:, 2 * HIDDEN : 3 * HIDDEN])
            o = jax.nn.sigmoid(gates[:, 3 * HIDDEN :])
            c = f * c + i * g
            h = o * jnp.tanh(c)
            hs_scr[pl.ds(b * SEQ_PAD, SEQ_PAD), :] = h
            return (h, c)

        init = (
            jnp.zeros((SEQ_PAD, HIDDEN), jnp.float32),
            jnp.zeros((SEQ_PAD, HIDDEN), jnp.float32),
        )
        lax.fori_loop(0, B, step, init)

    a = hs_scr[...].astype(jnp.bfloat16)
    w = wfc_ref[...].astype(jnp.bfloat16)
    acc = lax.dot_general(a, w, _NT, preferred_element_type=jnp.float32)
    acc = acc + bfc_ref[...]
    o_ref[...] = acc.reshape(B, SEQ_PAD, acc.shape[-1])[:, :SEQ, :]


def kernel(features, captions, emb_table, W_ih, W_hh, b_ih, b_hh, W_fc, b_fc):
    idx = captions.reshape(-1).astype(jnp.int32)
    idx_pad = jnp.pad(idx, (0, IDX_PAD - B * T))
    emb = _sc_gather(emb_table, idx_pad)[: B * T].reshape(B, T, EMBED)

    x = jnp.concatenate([features[:, None, :], emb], axis=1)
    x = jnp.pad(x, ((0, 0), (0, SEQ_PAD - SEQ), (0, 0)))
    x = x.reshape(B * SEQ_PAD, EMBED)

    c0 = lambda i: (0, 0)
    out = pl.pallas_call(
        _fused_body,
        grid=(N_VBLK,),
        in_specs=[
            pl.BlockSpec((B * SEQ_PAD, EMBED), c0),
            pl.BlockSpec((4 * HIDDEN, EMBED), c0),
            pl.BlockSpec((4 * HIDDEN, HIDDEN), c0),
            pl.BlockSpec((1, 4 * HIDDEN), c0),
            pl.BlockSpec((1, 4 * HIDDEN), c0),
            pl.BlockSpec((V_BLK, HIDDEN), lambda i: (i, 0)),
            pl.BlockSpec((1, V_BLK), lambda i: (0, i)),
        ],
        out_specs=pl.BlockSpec((B, SEQ, V_BLK), lambda i: (0, 0, i)),
        out_shape=jax.ShapeDtypeStruct((B, SEQ, VOCAB), jnp.float32),
        scratch_shapes=[pltpu.VMEM((B * SEQ_PAD, HIDDEN), jnp.float32)],
    )(x, W_ih, W_hh, b_ih.reshape(1, -1), b_hh.reshape(1, -1), W_fc,
      b_fc.reshape(1, VOCAB))
    return out
